# baseline (device time: 27452 ns/iter reference)
import jax
import jax.numpy as jnp
from jax import lax
from jax.experimental import pallas as pl
from jax.experimental.pallas import tpu as pltpu

N_DEV = 4
N_HOPS = 2 * (N_DEV - 1)
STRIPS = 4
N_RINGS = 2 * STRIPS


def kernel(x, w_mat):
    m, k_per = x.shape
    _, n = w_mat.shape
    mc = m // N_DEV
    nq = n // N_RINGS

    def body(x_ref, w_ref, out_ref, partial, stage, recv, ostage, xv, wv,
             send_sems, recv_sems, osems, isems):
        my = lax.axis_index("i")
        left = (my + (N_DEV - 1)) % N_DEV
        right = (my + 1) % N_DEV

        def rows(c):
            return pl.ds((c % N_DEV) * mc, mc)

        ring_dir = [r % 2 for r in range(N_RINGS)]
        ring_cols = [
            pl.ds((r % 2) * (n // 2) + (r // 2) * nq, nq)
            for r in range(N_RINGS)
        ]
        rc_tab = [
            [my - 1, my - 2, my - 3, my, my - 1, my - 2],
            [my + 1, my + 2, my + 3, my, my + 1, my + 2],
        ]

        def mk(r, h, src):
            return pltpu.make_async_remote_copy(
                src_ref=src, dst_ref=recv.at[r, h],
                send_sem=send_sems.at[r, h],
                recv_sem=recv_sems.at[r, h],
                device_id=(right if ring_dir[r] == 0 else left,),
                device_id_type=pl.DeviceIdType.MESH,
            )

        cpx = pltpu.make_async_copy(x_ref, xv, isems.at[0])
        cpw = pltpu.make_async_copy(w_ref, wv, isems.at[1])
        cpx.start()
        cpw.start()

        barrier_sem = pltpu.get_barrier_semaphore()
        for nbr in (left, right):
            pl.semaphore_signal(
                barrier_sem, inc=1,
                device_id=(nbr,), device_id_type=pl.DeviceIdType.MESH,
            )

        cpx.wait()
        cpw.wait()
        w_bf = wv[:, :].astype(jnp.bfloat16)

        partial[rows(my), :] = jnp.dot(
            xv[rows(my), :].astype(jnp.bfloat16), w_bf,
            preferred_element_type=jnp.float32,
        ).astype(jnp.bfloat16)

        pl.semaphore_wait(barrier_sem, 2)

        out_cps = [None] * N_RINGS
        descs = [[None] * N_HOPS for _ in range(N_RINGS)]
        for r in range(N_RINGS):
            descs[r][0] = mk(r, 0, partial.at[rows(my), ring_cols[r]])
            descs[r][0].start()

        for j in range(1, N_DEV):
            partial[rows(my + j), :] = jnp.dot(
                xv[rows(my + j), :].astype(jnp.bfloat16), w_bf,
                preferred_element_type=jnp.float32,
            ).astype(jnp.bfloat16)

        for h in range(N_DEV - 1):
            k = (h + 1) % 2
            own_accs = []
            for r in range(N_RINGS):
                rc = rc_tab[ring_dir[r]][h]
                descs[r][h].wait_recv()
                acc = partial[rows(rc), ring_cols[r]] + recv[r, h]
                if h >= 1:
                    descs[r][h - 1].wait_send()
                stage[r, k] = acc
                descs[r][h + 1] = mk(r, h + 1, stage.at[r, k])
                descs[r][h + 1].start()
                if h == N_DEV - 2:
                    own_accs.append((rc, r, acc))
            for rc, r, acc in own_accs:
                ostage[r] = acc.astype(jnp.float32)
                cp = pltpu.make_async_copy(
                    ostage.at[r], out_ref.at[rows(rc), ring_cols[r]],
                    osems.at[r],
                )
                cp.start()
                out_cps[r] = cp

        for h in range(N_DEV - 1, N_HOPS):
            stores = []
            for r in range(N_RINGS):
                descs[r][h].wait_recv()
                if h + 1 < N_HOPS:
                    descs[r][h + 1] = mk(r, h + 1, recv.at[r, h])
                    descs[r][h + 1].start()
                stores.append(r)
            for r in stores:
                rc = rc_tab[ring_dir[r]][h]
                out_cps[r].wait()
                ostage[r] = recv[r, h].astype(jnp.float32)
                cp = pltpu.make_async_copy(
                    ostage.at[r], out_ref.at[rows(rc), ring_cols[r]],
                    osems.at[r],
                )
                cp.start()
                out_cps[r] = cp

        for r in range(N_RINGS):
            for h in range(2, N_HOPS):
                descs[r][h].wait_send()
            out_cps[r].wait()

    return pl.pallas_call(
        body,
        out_shape=jax.ShapeDtypeStruct((m, n), jnp.float32),
        in_specs=[
            pl.BlockSpec(memory_space=pltpu.MemorySpace.HBM),
            pl.BlockSpec(memory_space=pltpu.MemorySpace.HBM),
        ],
        out_specs=pl.BlockSpec(memory_space=pltpu.MemorySpace.HBM),
        scratch_shapes=[
            pltpu.VMEM((m, n), jnp.bfloat16),
            pltpu.VMEM((N_RINGS, 2, mc, nq), jnp.bfloat16),
            pltpu.VMEM((N_RINGS, N_HOPS, mc, nq), jnp.bfloat16),
            pltpu.VMEM((N_RINGS, mc, nq), jnp.float32),
            pltpu.VMEM((m, k_per), jnp.float32),
            pltpu.VMEM((k_per, n), jnp.float32),
            pltpu.SemaphoreType.DMA((N_RINGS, N_HOPS)),
            pltpu.SemaphoreType.DMA((N_RINGS, N_HOPS)),
            pltpu.SemaphoreType.DMA((N_RINGS,)),
            pltpu.SemaphoreType.DMA((2,)),
        ],
        compiler_params=pltpu.CompilerParams(collective_id=0),
    )(x, w_mat)


# device time: 27099 ns/iter; 1.0130x vs baseline; 1.0130x over previous
import jax
import jax.numpy as jnp
from jax import lax
from jax.experimental import pallas as pl
from jax.experimental.pallas import tpu as pltpu

N_DEV = 4
N_HOPS = 2 * (N_DEV - 1)
STRIPS = 4
N_RINGS = 2 * STRIPS


def kernel(x, w_mat):
    m, _ = x.shape
    _, n = w_mat.shape
    mc = m // N_DEV
    nq = n // N_RINGS

    def body(x_ref, w_ref, out_ref, partial, stage, recv, send_sems,
             recv_sems):
        my = lax.axis_index("i")
        left = (my + (N_DEV - 1)) % N_DEV
        right = (my + 1) % N_DEV

        def rows(c):
            return pl.ds((c % N_DEV) * mc, mc)

        ring_dir = [r % 2 for r in range(N_RINGS)]
        ring_cols = [
            pl.ds((r % 2) * (n // 2) + (r // 2) * nq, nq)
            for r in range(N_RINGS)
        ]
        rc_tab = [
            [my - 1, my - 2, my - 3, my, my - 1, my - 2],
            [my + 1, my + 2, my + 3, my, my + 1, my + 2],
        ]

        def mk(r, h, src):
            return pltpu.make_async_remote_copy(
                src_ref=src, dst_ref=recv.at[r, h],
                send_sem=send_sems.at[r, h],
                recv_sem=recv_sems.at[r, h],
                device_id=(right if ring_dir[r] == 0 else left,),
                device_id_type=pl.DeviceIdType.MESH,
            )

        barrier_sem = pltpu.get_barrier_semaphore()
        for nbr in (left, right):
            pl.semaphore_signal(
                barrier_sem, inc=1,
                device_id=(nbr,), device_id_type=pl.DeviceIdType.MESH,
            )

        w_bf = w_ref[:, :].astype(jnp.bfloat16)

        partial[rows(my), :] = jnp.dot(
            x_ref[rows(my), :].astype(jnp.bfloat16), w_bf,
            preferred_element_type=jnp.float32,
        ).astype(jnp.bfloat16)

        pl.semaphore_wait(barrier_sem, 2)

        descs = [[None] * N_HOPS for _ in range(N_RINGS)]
        for r in range(N_RINGS):
            descs[r][0] = mk(r, 0, partial.at[rows(my), ring_cols[r]])
            descs[r][0].start()

        for j in range(1, N_DEV):
            partial[rows(my + j), :] = jnp.dot(
                x_ref[rows(my + j), :].astype(jnp.bfloat16), w_bf,
                preferred_element_type=jnp.float32,
            ).astype(jnp.bfloat16)

        for h in range(N_DEV - 1):
            k = (h + 1) % 2
            own_accs = []
            for r in range(N_RINGS):
                rc = rc_tab[ring_dir[r]][h]
                descs[r][h].wait_recv()
                acc = partial[rows(rc), ring_cols[r]] + recv[r, h]
                if h >= 1:
                    descs[r][h - 1].wait_send()
                stage[r, k] = acc
                descs[r][h + 1] = mk(r, h + 1, stage.at[r, k])
                descs[r][h + 1].start()
                if h == N_DEV - 2:
                    own_accs.append((rc, r, acc))
            for rc, r, acc in own_accs:
                out_ref[rows(rc), ring_cols[r]] = acc.astype(jnp.float32)

        for h in range(N_DEV - 1, N_HOPS):
            stores = []
            for r in range(N_RINGS):
                descs[r][h].wait_recv()
                if h + 1 < N_HOPS:
                    descs[r][h + 1] = mk(r, h + 1, recv.at[r, h])
                    descs[r][h + 1].start()
                stores.append(r)
            for r in stores:
                rc = rc_tab[ring_dir[r]][h]
                out_ref[rows(rc), ring_cols[r]] = recv[r, h].astype(
                    jnp.float32
                )

        for r in range(N_RINGS):
            for h in range(2, N_HOPS):
                descs[r][h].wait_send()

    return pl.pallas_call(
        body,
        out_shape=jax.ShapeDtypeStruct((m, n), jnp.float32),
        in_specs=[
            pl.BlockSpec(memory_space=pltpu.VMEM),
            pl.BlockSpec(memory_space=pltpu.VMEM),
        ],
        out_specs=pl.BlockSpec(memory_space=pltpu.VMEM),
        scratch_shapes=[
            pltpu.VMEM((m, n), jnp.bfloat16),
            pltpu.VMEM((N_RINGS, 2, mc, nq), jnp.bfloat16),
            pltpu.VMEM((N_RINGS, N_HOPS, mc, nq), jnp.bfloat16),
            pltpu.SemaphoreType.DMA((N_RINGS, N_HOPS)),
            pltpu.SemaphoreType.DMA((N_RINGS, N_HOPS)),
        ],
        compiler_params=pltpu.CompilerParams(collective_id=0),
    )(x, w_mat)


# device time: 27008 ns/iter; 1.0164x vs baseline; 1.0034x over previous
import jax
import jax.numpy as jnp
from jax import lax
from jax.experimental import pallas as pl
from jax.experimental.pallas import tpu as pltpu

N_DEV = 4
N_HOPS = 2 * (N_DEV - 1)
STRIPS = 4
N_RINGS = 2 * STRIPS


def kernel(x, w_mat):
    m, _ = x.shape
    _, n = w_mat.shape
    mc = m // N_DEV
    nq = n // N_RINGS

    def body(x_ref, w_ref, out_ref, partial, stage, recv, send_sems,
             recv_sems):
        my = lax.axis_index("i")
        left = (my + (N_DEV - 1)) % N_DEV
        right = (my + 1) % N_DEV

        def rows(c):
            return pl.ds((c % N_DEV) * mc, mc)

        ring_dir = [r % 2 for r in range(N_RINGS)]
        ring_cols = [
            pl.ds((r % 2) * (n // 2) + (r // 2) * nq, nq)
            for r in range(N_RINGS)
        ]
        rc_tab = [
            [my - 1, my - 2, my - 3, my, my - 1, my - 2],
            [my + 1, my + 2, my + 3, my, my + 1, my + 2],
        ]

        def mk(r, h, src):
            return pltpu.make_async_remote_copy(
                src_ref=src, dst_ref=recv.at[r, h],
                send_sem=send_sems.at[r, h],
                recv_sem=recv_sems.at[r, h],
                device_id=(right if ring_dir[r] == 0 else left,),
                device_id_type=pl.DeviceIdType.MESH,
            )

        barrier_sem = pltpu.get_barrier_semaphore()
        for nbr in (left, right):
            pl.semaphore_signal(
                barrier_sem, inc=1,
                device_id=(nbr,), device_id_type=pl.DeviceIdType.MESH,
            )

        w_bf = w_ref[:, :].astype(jnp.bfloat16)

        partial[rows(my), :] = jnp.dot(
            x_ref[rows(my), :].astype(jnp.bfloat16), w_bf,
            preferred_element_type=jnp.float32,
        ).astype(jnp.bfloat16)

        pl.semaphore_wait(barrier_sem, 2)

        descs = [[None] * N_HOPS for _ in range(N_RINGS)]
        for r in range(N_RINGS):
            descs[r][0] = mk(r, 0, partial.at[rows(my), ring_cols[r]])
            descs[r][0].start()

        for j in range(1, N_DEV):
            partial[rows(my + j), :] = jnp.dot(
                x_ref[rows(my + j), :].astype(jnp.bfloat16), w_bf,
                preferred_element_type=jnp.float32,
            ).astype(jnp.bfloat16)

        for h in range(N_DEV - 1):
            k = (h + 1) % 2
            own_accs = []
            for r in range(N_RINGS):
                rc = rc_tab[ring_dir[r]][h]
                descs[r][h].wait_recv()
                acc = partial[rows(rc), ring_cols[r]] + recv[r, h]
                if h >= 1:
                    descs[r][h - 1].wait_send()
                stage[r, k] = acc
                descs[r][h + 1] = mk(r, h + 1, stage.at[r, k])
                descs[r][h + 1].start()
                if h == N_DEV - 2:
                    own_accs.append((rc, r, acc))
            for rc, r, acc in own_accs:
                out_ref[rows(rc), ring_cols[r]] = acc.astype(jnp.float32)

        for h in range(N_DEV - 1, N_HOPS):
            last = h + 1 >= N_HOPS
            stores = []
            for r in range(N_RINGS):
                rc = rc_tab[ring_dir[r]][h]
                descs[r][h].wait_recv()
                if not last:
                    descs[r][h + 1] = mk(r, h + 1, recv.at[r, h])
                    descs[r][h + 1].start()
                    stores.append(r)
                else:
                    out_ref[rows(rc), ring_cols[r]] = recv[r, h].astype(
                        jnp.float32
                    )
            for r in stores:
                rc = rc_tab[ring_dir[r]][h]
                out_ref[rows(rc), ring_cols[r]] = recv[r, h].astype(
                    jnp.float32
                )

        for r in range(N_RINGS):
            for h in range(2, N_HOPS):
                descs[r][h].wait_send()

    return pl.pallas_call(
        body,
        out_shape=jax.ShapeDtypeStruct((m, n), jnp.float32),
        in_specs=[
            pl.BlockSpec(memory_space=pltpu.VMEM),
            pl.BlockSpec(memory_space=pltpu.VMEM),
        ],
        out_specs=pl.BlockSpec(memory_space=pltpu.VMEM),
        scratch_shapes=[
            pltpu.VMEM((m, n), jnp.bfloat16),
            pltpu.VMEM((N_RINGS, 2, mc, nq), jnp.bfloat16),
            pltpu.VMEM((N_RINGS, N_HOPS, mc, nq), jnp.bfloat16),
            pltpu.SemaphoreType.DMA((N_RINGS, N_HOPS)),
            pltpu.SemaphoreType.DMA((N_RINGS, N_HOPS)),
        ],
        compiler_params=pltpu.CompilerParams(collective_id=0),
    )(x, w_mat)
